# baseline (speedup 1.0000x reference)
import jax
import jax.numpy as jnp
from jax import lax
from jax.experimental import pallas as pl
from jax.experimental.pallas import tpu as pltpu
from jax.experimental.pallas import tpu_sc as plsc

N = 10000
E = 320000
D = 128
T = 101

NC = 2
NS = 16
NW = NC * NS
C = 64
SUP = 2
NCHUNK = E // C
K_FULL = NCHUNK // NW
K_REM = NCHUNK % NW
NSUP = K_FULL // SUP
NP = 10240
RPT = NP // NS
DB = D // 16


def _sc_body(send_h, recv_h, type_h, vproj_h, vtypes_h, b_h,
             out_h, deg_h,
             accum_s, degacc_s,
             types_v, b_v,
             sidx_v, tidx_v, ridx_v, ridxc_v, ridx8_v, iota_v, onesp_v,
             grows0_v, grows1_v, sem, msgsem):
    NSTR = 4
    CS = C // NSTR
    c = lax.axis_index("c")
    s = lax.axis_index("s")
    wid = s * NC + c

    pltpu.sync_copy(vtypes_h, types_v)
    pltpu.sync_copy(b_h, b_v)
    bvecs = [b_v[pl.ds(db * 16, 16)] for db in range(DB)]

    zero16 = jnp.zeros((16,), jnp.float32)
    one16 = jnp.ones((16,), jnp.float32)
    lanes = lax.iota(jnp.int32, 16)

    def _init_row(i, _):
        for db in range(DB):
            grows0_v[i, pl.ds(db * 16, 16)] = zero16
            onesp_v[i, pl.ds(db * 16, 16)] = zero16
        return 0
    lax.fori_loop(0, C, _init_row, 0)

    base = s * RPT

    def _fill_iota(b0):
        for gi in range(C // 16):
            iota_v[pl.ds(gi * 16, 16)] = lanes + (b0 + gi * 16)

    def _zero(j, _):
        _fill_iota(base + j * C)
        pltpu.sync_copy(grows0_v, accum_s.at[iota_v])
        return 0
    lax.fori_loop(0, RPT // C, _zero, 0)

    def _zero_deg(i):
        _fill_iota(i * C)
        pltpu.sync_copy(grows0_v, degacc_s.at[iota_v])
    _zero_deg(s)

    @pl.when(s < (NP // 8) // C - NS)
    def _zero_deg_extra():
        _zero_deg(NS + s)
    plsc.subcore_barrier()

    edge0 = (wid * K_FULL + jnp.minimum(wid, K_REM)) * C
    grows = (grows0_v, grows1_v)

    def _compute(j):
        gbuf = grows[j % 2]

        @plsc.parallel_loop(0, C // 16, unroll=2)
        def _group(gi):
            tvec = tidx_v[pl.ds(j * C + gi * 16, 16)]
            rvec = ridx_v[pl.ds(j * C + gi * 16, 16)]
            ridxc_v[j % 2, pl.ds(gi * 16, 16)] = rvec
            ridx8_v[pl.ds(gi * 16, 16)] = lax.shift_right_logical(rvec, 3)
            for l in range(16):
                t = tvec[l]
                slot = (rvec[l] & 7) * 16
                e = gi * 16 + l
                onesp_v[e, pl.ds(slot, 16)] = one16
                for db in range(DB):
                    sv = gbuf[e, pl.ds(db * 16, 16)]
                    tv = types_v[t, pl.ds(db * 16, 16)]
                    gbuf[e, pl.ds(db * 16, 16)] = jnp.maximum(
                        sv * tv + bvecs[db], 0.0)

    def _wait_msg(j):
        pltpu.make_async_copy(grows[j % 2],
                              accum_s.at[ridxc_v.at[j % 2]], msgsem).wait()

    def _deg_scatter(j):
        pltpu.sync_copy(onesp_v, degacc_s.at[ridx8_v], add=True)

        @plsc.parallel_loop(0, C // 16, unroll=2)
        def _unset(gi):
            rvec = ridx_v[pl.ds(j * C + gi * 16, 16)]
            for l in range(16):
                slot = (rvec[l] & 7) * 16
                onesp_v[gi * 16 + l, pl.ds(slot, 16)] = zero16

    def _super(k2, _):
        off = edge0 + k2 * (SUP * C)
        pltpu.sync_copy(send_h.at[pl.ds(off, SUP * C)], sidx_v)
        pltpu.sync_copy(type_h.at[pl.ds(off, SUP * C)], tidx_v)
        pltpu.sync_copy(recv_h.at[pl.ds(off, SUP * C)], ridx_v)
        cps = [pltpu.async_copy(
            vproj_h.at[sidx_v.at[pl.ds(n * CS, CS)]],
            grows0_v.at[pl.ds(n * CS, CS)], sem) for n in range(NSTR)]
        for j in range(SUP):
            for cp in cps:
                cp.wait()
            if j == 0:
                @pl.when(k2 > 0)
                def _():
                    _wait_msg(1)
            else:
                _wait_msg(0)
            if j + 1 < SUP:
                cps = [pltpu.async_copy(
                    vproj_h.at[sidx_v.at[pl.ds((j + 1) * C + n * CS, CS)]],
                    grows[(j + 1) % 2].at[pl.ds(n * CS, CS)], sem)
                    for n in range(NSTR)]
            _compute(j)
            pltpu.async_copy(grows[j % 2],
                             accum_s.at[ridxc_v.at[j % 2]], msgsem,
                             add=True)
            _deg_scatter(j)
        return 0

    lax.fori_loop(0, NSUP, _super, 0)

    @pl.when(wid < K_REM)
    def _tail():
        _wait_msg(1)
        off = edge0 + K_FULL * C
        pltpu.sync_copy(send_h.at[pl.ds(off, C)], sidx_v.at[pl.ds(0, C)])
        pltpu.sync_copy(type_h.at[pl.ds(off, C)], tidx_v.at[pl.ds(0, C)])
        pltpu.sync_copy(recv_h.at[pl.ds(off, C)], ridx_v.at[pl.ds(0, C)])
        pltpu.async_copy(vproj_h.at[sidx_v.at[pl.ds(0, C)]],
                         grows0_v, sem).wait()
        _compute(0)
        pltpu.sync_copy(grows0_v, accum_s.at[ridxc_v.at[0]], add=True)
        _deg_scatter(0)

    @pl.when(wid >= K_REM)
    def _drain():
        _wait_msg(1)

    plsc.subcore_barrier()

    def _writeout(j, _):
        _fill_iota(base + j * C)
        pltpu.sync_copy(accum_s.at[iota_v], grows0_v)
        pltpu.sync_copy(grows0_v, out_h.at[c, pl.ds(base + j * C, C)])
        return 0
    lax.fori_loop(0, RPT // C, _writeout, 0)

    def _writeout_deg(i):
        _fill_iota(i * C)
        pltpu.sync_copy(degacc_s.at[iota_v], grows0_v)
        pltpu.sync_copy(grows0_v, deg_h.at[c, pl.ds(i * C, C)])
    _writeout_deg(s)

    @pl.when(s < (NP // 8) // C - NS)
    def _writeout_deg_extra():
        _writeout_deg(NS + s)


_sc_accumulate = pl.kernel(
    _sc_body,
    out_type=(
        jax.ShapeDtypeStruct((NC, NP, D), jnp.float32),
        jax.ShapeDtypeStruct((NC, NP // 8, D), jnp.float32),
    ),
    mesh=plsc.VectorSubcoreMesh(core_axis_name="c", subcore_axis_name="s",
                                num_cores=NC, num_subcores=NS),
    scratch_types=(
        pltpu.VMEM_SHARED((NP, D), jnp.float32),
        pltpu.VMEM_SHARED((NP // 8, D), jnp.float32),
        pltpu.VMEM((T, D), jnp.float32),
        pltpu.VMEM((D,), jnp.float32),
        pltpu.VMEM((SUP * C,), jnp.int32),
        pltpu.VMEM((SUP * C,), jnp.int32),
        pltpu.VMEM((SUP * C,), jnp.int32),
        pltpu.VMEM((2, C), jnp.int32),
        pltpu.VMEM((C,), jnp.int32),
        pltpu.VMEM((C,), jnp.int32),
        pltpu.VMEM((C, D), jnp.float32),
        pltpu.VMEM((C, D), jnp.float32),
        pltpu.VMEM((C, D), jnp.float32),
        pltpu.SemaphoreType.DMA,
        pltpu.SemaphoreType.DMA,
    ),
)


def _finalize_body(msg_ref, deg_ref, out_ref):
    p = msg_ref[0] + msg_ref[1]
    dsum = deg_ref[:, 0:1] + deg_ref[:, 1:2]
    out_ref[...] = p * (1.0 / jnp.maximum(dsum, 1.0))


_ROWS_B = 1024

_finalize = pl.pallas_call(
    _finalize_body,
    grid=(NP // _ROWS_B,),
    in_specs=[
        pl.BlockSpec((NC, _ROWS_B, D), lambda i: (0, i, 0)),
        pl.BlockSpec((_ROWS_B, NC), lambda i: (i, 0)),
    ],
    out_specs=pl.BlockSpec((_ROWS_B, D), lambda i: (i, 0)),
    out_shape=jax.ShapeDtypeStruct((NP, D), jnp.float32),
)


def kernel(sender_indices, receiver_indices, type_indices,
           V_proj_sender, V_types, B_message):
    msg_p, deg_p = _sc_accumulate(sender_indices, receiver_indices,
                                  type_indices, V_proj_sender, V_types,
                                  B_message)
    deg_cols = deg_p.reshape(NC, NP // 8, 8, 16)[:, :, :, 0].reshape(NC, NP).T
    return _finalize(msg_p, deg_cols)[:N]


# speedup vs baseline: 1.3226x; 1.3226x over previous
import jax
import jax.numpy as jnp
from jax import lax
from jax.experimental import pallas as pl
from jax.experimental.pallas import tpu as pltpu
from jax.experimental.pallas import tpu_sc as plsc

N = 10000
E = 320000
D = 128
T = 101

NC = 2
NS = 16
NW = NC * NS
C = 64
SUP = 2
NCHUNK = E // C
K_FULL = NCHUNK // NW
K_REM = NCHUNK % NW
NSUP = K_FULL // SUP
NP = 10240
RPT = NP // NS
DB = D // 16


def _sc_body(send_h, recv_h, type_h, vproj_h, vtypes_h, b_h,
             out_h, deg_h,
             accum_s, degacc_s,
             types_v, b_v,
             sidx_v, tidx_v, ridx_v, ridxc_v, ridx8_v, iota_v, onesp_v,
             grows0_v, grows1_v, sem, msgsem):
    NSTR = 4
    CS = C // NSTR
    c = lax.axis_index("c")
    s = lax.axis_index("s")
    wid = s * NC + c

    pltpu.sync_copy(vtypes_h, types_v)
    pltpu.sync_copy(b_h, b_v)
    bvecs = [b_v[pl.ds(db * 16, 16)] for db in range(DB)]

    zero16 = jnp.zeros((16,), jnp.float32)
    one16 = jnp.ones((16,), jnp.float32)
    lanes = lax.iota(jnp.int32, 16)

    def _init_row(i, _):
        for db in range(DB):
            grows0_v[i, pl.ds(db * 16, 16)] = zero16
            onesp_v[i, pl.ds(db * 16, 16)] = zero16
        return 0
    lax.fori_loop(0, C, _init_row, 0)

    base = s * RPT

    def _fill_iota(b0):
        for gi in range(C // 16):
            iota_v[pl.ds(gi * 16, 16)] = lanes + (b0 + gi * 16)

    def _zero(j, _):
        _fill_iota(base + j * C)
        pltpu.sync_copy(grows0_v, accum_s.at[iota_v])
        return 0
    lax.fori_loop(0, RPT // C, _zero, 0)

    def _zero_deg(i):
        _fill_iota(i * C)
        pltpu.sync_copy(grows0_v, degacc_s.at[iota_v])
    _zero_deg(s)

    @pl.when(s < (NP // 8) // C - NS)
    def _zero_deg_extra():
        _zero_deg(NS + s)
    plsc.subcore_barrier()

    edge0 = (wid * K_FULL + jnp.minimum(wid, K_REM)) * C
    grows = (grows0_v, grows1_v)

    def _compute(j):
        gbuf = grows[j % 2]

        @plsc.parallel_loop(0, C // 16, unroll=1)
        def _group(gi):
            tvec = tidx_v[pl.ds(j * C + gi * 16, 16)]
            rvec = ridx_v[pl.ds(j * C + gi * 16, 16)]
            ridxc_v[j % 2, pl.ds(gi * 16, 16)] = rvec
            ridx8_v[pl.ds(gi * 16, 16)] = lax.shift_right_logical(rvec, 3)
            for l in range(16):
                t = tvec[l]
                slot = (rvec[l] & 7) * 16
                e = gi * 16 + l
                onesp_v[e, pl.ds(slot, 16)] = one16
                for db in range(DB):
                    sv = gbuf[e, pl.ds(db * 16, 16)]
                    tv = types_v[t, pl.ds(db * 16, 16)]
                    gbuf[e, pl.ds(db * 16, 16)] = jnp.maximum(
                        sv * tv + bvecs[db], 0.0)

    def _wait_msg(j):
        pltpu.make_async_copy(grows[j % 2],
                              accum_s.at[ridxc_v.at[j % 2]], msgsem).wait()

    def _deg_scatter(j):
        pltpu.sync_copy(onesp_v, degacc_s.at[ridx8_v], add=True)

        @plsc.parallel_loop(0, C // 16, unroll=1)
        def _unset(gi):
            rvec = ridx_v[pl.ds(j * C + gi * 16, 16)]
            for l in range(16):
                slot = (rvec[l] & 7) * 16
                onesp_v[gi * 16 + l, pl.ds(slot, 16)] = zero16

    def _super(k2, _):
        off = edge0 + k2 * (SUP * C)
        pltpu.sync_copy(send_h.at[pl.ds(off, SUP * C)], sidx_v)
        pltpu.sync_copy(type_h.at[pl.ds(off, SUP * C)], tidx_v)
        pltpu.sync_copy(recv_h.at[pl.ds(off, SUP * C)], ridx_v)
        cps = [pltpu.async_copy(
            vproj_h.at[sidx_v.at[pl.ds(n * CS, CS)]],
            grows0_v.at[pl.ds(n * CS, CS)], sem) for n in range(NSTR)]
        for j in range(SUP):
            for cp in cps:
                cp.wait()
            if j == 0:
                @pl.when(k2 > 0)
                def _():
                    _wait_msg(1)
            else:
                _wait_msg(0)
            if j + 1 < SUP:
                cps = [pltpu.async_copy(
                    vproj_h.at[sidx_v.at[pl.ds((j + 1) * C + n * CS, CS)]],
                    grows[(j + 1) % 2].at[pl.ds(n * CS, CS)], sem)
                    for n in range(NSTR)]
            _compute(j)
            pltpu.async_copy(grows[j % 2],
                             accum_s.at[ridxc_v.at[j % 2]], msgsem,
                             add=True)
            _deg_scatter(j)
        return 0

    lax.fori_loop(0, NSUP, _super, 0)

    @pl.when(wid < K_REM)
    def _tail():
        _wait_msg(1)
        off = edge0 + K_FULL * C
        pltpu.sync_copy(send_h.at[pl.ds(off, C)], sidx_v.at[pl.ds(0, C)])
        pltpu.sync_copy(type_h.at[pl.ds(off, C)], tidx_v.at[pl.ds(0, C)])
        pltpu.sync_copy(recv_h.at[pl.ds(off, C)], ridx_v.at[pl.ds(0, C)])
        pltpu.async_copy(vproj_h.at[sidx_v.at[pl.ds(0, C)]],
                         grows0_v, sem).wait()
        _compute(0)
        pltpu.sync_copy(grows0_v, accum_s.at[ridxc_v.at[0]], add=True)
        _deg_scatter(0)

    @pl.when(wid >= K_REM)
    def _drain():
        _wait_msg(1)

    plsc.subcore_barrier()

    def _writeout(j, _):
        _fill_iota(base + j * C)
        pltpu.sync_copy(accum_s.at[iota_v], grows0_v)
        pltpu.sync_copy(grows0_v, out_h.at[c, pl.ds(base + j * C, C)])
        return 0
    lax.fori_loop(0, RPT // C, _writeout, 0)

    def _writeout_deg(i):
        _fill_iota(i * C)
        pltpu.sync_copy(degacc_s.at[iota_v], grows0_v)
        pltpu.sync_copy(grows0_v, deg_h.at[c, pl.ds(i * C, C)])
    _writeout_deg(s)

    @pl.when(s < (NP // 8) // C - NS)
    def _writeout_deg_extra():
        _writeout_deg(NS + s)


_sc_accumulate = pl.kernel(
    _sc_body,
    out_type=(
        jax.ShapeDtypeStruct((NC, NP, D), jnp.float32),
        jax.ShapeDtypeStruct((NC, NP // 8, D), jnp.float32),
    ),
    mesh=plsc.VectorSubcoreMesh(core_axis_name="c", subcore_axis_name="s",
                                num_cores=NC, num_subcores=NS),
    scratch_types=(
        pltpu.VMEM_SHARED((NP, D), jnp.float32),
        pltpu.VMEM_SHARED((NP // 8, D), jnp.float32),
        pltpu.VMEM((T, D), jnp.float32),
        pltpu.VMEM((D,), jnp.float32),
        pltpu.VMEM((SUP * C,), jnp.int32),
        pltpu.VMEM((SUP * C,), jnp.int32),
        pltpu.VMEM((SUP * C,), jnp.int32),
        pltpu.VMEM((2, C), jnp.int32),
        pltpu.VMEM((C,), jnp.int32),
        pltpu.VMEM((C,), jnp.int32),
        pltpu.VMEM((C, D), jnp.float32),
        pltpu.VMEM((C, D), jnp.float32),
        pltpu.VMEM((C, D), jnp.float32),
        pltpu.SemaphoreType.DMA,
        pltpu.SemaphoreType.DMA,
    ),
)


def _finalize_body(msg_ref, deg_ref, out_ref):
    p = msg_ref[0] + msg_ref[1]
    dsum = deg_ref[:, 0:1] + deg_ref[:, 1:2]
    out_ref[...] = p * (1.0 / jnp.maximum(dsum, 1.0))


_ROWS_B = 1024

_finalize = pl.pallas_call(
    _finalize_body,
    grid=(NP // _ROWS_B,),
    in_specs=[
        pl.BlockSpec((NC, _ROWS_B, D), lambda i: (0, i, 0)),
        pl.BlockSpec((_ROWS_B, NC), lambda i: (i, 0)),
    ],
    out_specs=pl.BlockSpec((_ROWS_B, D), lambda i: (i, 0)),
    out_shape=jax.ShapeDtypeStruct((NP, D), jnp.float32),
)


def kernel(sender_indices, receiver_indices, type_indices,
           V_proj_sender, V_types, B_message):
    msg_p, deg_p = _sc_accumulate(sender_indices, receiver_indices,
                                  type_indices, V_proj_sender, V_types,
                                  B_message)
    deg_cols = deg_p.reshape(NC, NP // 8, 8, 16)[:, :, :, 0].reshape(NC, NP).T
    return _finalize(msg_p, deg_cols)[:N]


# speedup vs baseline: 1.3732x; 1.0383x over previous
import jax
import jax.numpy as jnp
from jax import lax
from jax.experimental import pallas as pl
from jax.experimental.pallas import tpu as pltpu
from jax.experimental.pallas import tpu_sc as plsc

N = 10000
E = 320000
D = 128
T = 101

NC = 2
NS = 16
NW = NC * NS
C = 64
SUP = 2
NCHUNK = E // C
K_FULL = NCHUNK // NW
K_REM = NCHUNK % NW
NSUP = K_FULL // SUP
NP = 10240
RPT = NP // NS
DB = D // 16


def _sc_body(send_h, recv_h, type_h, vproj_h, vtypes_h, b_h,
             out_h, deg_h,
             accum_s, degacc_s,
             types_v, b_v,
             sidx_v, tidx_v, ridx_v, ridxc_v, ridx8_v, iota_v, onesp_v,
             grows0_v, grows1_v, sem, msgsem):
    c = lax.axis_index("c")
    s = lax.axis_index("s")
    wid = s * NC + c

    pltpu.sync_copy(vtypes_h, types_v)
    pltpu.sync_copy(b_h, b_v)
    bvecs = [b_v[pl.ds(db * 16, 16)] for db in range(DB)]

    zero16 = jnp.zeros((16,), jnp.float32)
    one16 = jnp.ones((16,), jnp.float32)
    lanes = lax.iota(jnp.int32, 16)

    def _init_row(i, _):
        for db in range(DB):
            grows0_v[i, pl.ds(db * 16, 16)] = zero16
            onesp_v[i, pl.ds(db * 16, 16)] = zero16
        return 0
    lax.fori_loop(0, C, _init_row, 0)

    base = s * RPT

    def _fill_iota(b0):
        for gi in range(C // 16):
            iota_v[pl.ds(gi * 16, 16)] = lanes + (b0 + gi * 16)

    def _zero(j, _):
        _fill_iota(base + j * C)
        pltpu.sync_copy(grows0_v, accum_s.at[iota_v])
        return 0
    lax.fori_loop(0, RPT // C, _zero, 0)

    def _zero_deg(i):
        _fill_iota(i * C)
        pltpu.sync_copy(grows0_v, degacc_s.at[iota_v])
    _zero_deg(s)

    @pl.when(s < (NP // 8) // C - NS)
    def _zero_deg_extra():
        _zero_deg(NS + s)
    plsc.subcore_barrier()

    edge0 = (wid * K_FULL + jnp.minimum(wid, K_REM)) * C
    grows = (grows0_v, grows1_v)

    def _compute(j):
        gbuf = grows[j % 2]

        def _group(gi, _):
            tvec = tidx_v[pl.ds(j * C + gi * 16, 16)]
            rvec = ridx_v[pl.ds(j * C + gi * 16, 16)]
            ridxc_v[j % 2, pl.ds(gi * 16, 16)] = rvec
            ridx8_v[pl.ds(gi * 16, 16)] = lax.shift_right_logical(rvec, 3)
            for l in range(16):
                t = tvec[l]
                slot = (rvec[l] & 7) * 16
                e = gi * 16 + l
                onesp_v[e, pl.ds(slot, 16)] = one16
                for db in range(DB):
                    sv = gbuf[e, pl.ds(db * 16, 16)]
                    tv = types_v[t, pl.ds(db * 16, 16)]
                    gbuf[e, pl.ds(db * 16, 16)] = jnp.maximum(
                        sv * tv + bvecs[db], 0.0)
            return 0
        lax.fori_loop(0, C // 16, _group, 0)

    def _wait_msg(j):
        pltpu.make_async_copy(grows[j % 2],
                              accum_s.at[ridxc_v.at[j % 2]], msgsem).wait()

    def _deg_scatter(j):
        pltpu.sync_copy(onesp_v, degacc_s.at[ridx8_v], add=True)

        def _unset(gi, _):
            rvec = ridx_v[pl.ds(j * C + gi * 16, 16)]
            for l in range(16):
                slot = (rvec[l] & 7) * 16
                onesp_v[gi * 16 + l, pl.ds(slot, 16)] = zero16
            return 0
        lax.fori_loop(0, C // 16, _unset, 0)

    def _super(k2, _):
        off = edge0 + k2 * (SUP * C)
        pltpu.sync_copy(send_h.at[pl.ds(off, SUP * C)], sidx_v)
        pltpu.sync_copy(type_h.at[pl.ds(off, SUP * C)], tidx_v)
        pltpu.sync_copy(recv_h.at[pl.ds(off, SUP * C)], ridx_v)
        cp = pltpu.async_copy(vproj_h.at[sidx_v.at[pl.ds(0, C)]],
                              grows0_v, sem)
        for j in range(SUP):
            cp.wait()
            if j == 0:
                @pl.when(k2 > 0)
                def _():
                    _wait_msg(1)
            else:
                _wait_msg(0)
            if j + 1 < SUP:
                cp = pltpu.async_copy(
                    vproj_h.at[sidx_v.at[pl.ds((j + 1) * C, C)]],
                    grows[(j + 1) % 2], sem)
            _compute(j)
            pltpu.async_copy(grows[j % 2],
                             accum_s.at[ridxc_v.at[j % 2]], msgsem,
                             add=True)
            _deg_scatter(j)
        return 0

    lax.fori_loop(0, NSUP, _super, 0)

    @pl.when(wid < K_REM)
    def _tail():
        _wait_msg(1)
        off = edge0 + K_FULL * C
        pltpu.sync_copy(send_h.at[pl.ds(off, C)], sidx_v.at[pl.ds(0, C)])
        pltpu.sync_copy(type_h.at[pl.ds(off, C)], tidx_v.at[pl.ds(0, C)])
        pltpu.sync_copy(recv_h.at[pl.ds(off, C)], ridx_v.at[pl.ds(0, C)])
        pltpu.async_copy(vproj_h.at[sidx_v.at[pl.ds(0, C)]],
                         grows0_v, sem).wait()
        _compute(0)
        pltpu.sync_copy(grows0_v, accum_s.at[ridxc_v.at[0]], add=True)
        _deg_scatter(0)

    @pl.when(wid >= K_REM)
    def _drain():
        _wait_msg(1)

    plsc.subcore_barrier()

    def _writeout(j, _):
        _fill_iota(base + j * C)
        pltpu.sync_copy(accum_s.at[iota_v], grows0_v)
        pltpu.sync_copy(grows0_v, out_h.at[c, pl.ds(base + j * C, C)])
        return 0
    lax.fori_loop(0, RPT // C, _writeout, 0)

    def _writeout_deg(i):
        _fill_iota(i * C)
        pltpu.sync_copy(degacc_s.at[iota_v], grows0_v)
        pltpu.sync_copy(grows0_v, deg_h.at[c, pl.ds(i * C, C)])
    _writeout_deg(s)

    @pl.when(s < (NP // 8) // C - NS)
    def _writeout_deg_extra():
        _writeout_deg(NS + s)


_sc_accumulate = pl.kernel(
    _sc_body,
    out_type=(
        jax.ShapeDtypeStruct((NC, NP, D), jnp.float32),
        jax.ShapeDtypeStruct((NC, NP // 8, D), jnp.float32),
    ),
    mesh=plsc.VectorSubcoreMesh(core_axis_name="c", subcore_axis_name="s",
                                num_cores=NC, num_subcores=NS),
    scratch_types=(
        pltpu.VMEM_SHARED((NP, D), jnp.float32),
        pltpu.VMEM_SHARED((NP // 8, D), jnp.float32),
        pltpu.VMEM((T, D), jnp.float32),
        pltpu.VMEM((D,), jnp.float32),
        pltpu.VMEM((SUP * C,), jnp.int32),
        pltpu.VMEM((SUP * C,), jnp.int32),
        pltpu.VMEM((SUP * C,), jnp.int32),
        pltpu.VMEM((2, C), jnp.int32),
        pltpu.VMEM((C,), jnp.int32),
        pltpu.VMEM((C,), jnp.int32),
        pltpu.VMEM((C, D), jnp.float32),
        pltpu.VMEM((C, D), jnp.float32),
        pltpu.VMEM((C, D), jnp.float32),
        pltpu.SemaphoreType.DMA,
        pltpu.SemaphoreType.DMA,
    ),
)


def _finalize_body(msg_ref, deg_ref, out_ref):
    p = msg_ref[0] + msg_ref[1]
    dsum = deg_ref[:, 0:1] + deg_ref[:, 1:2]
    out_ref[...] = p * (1.0 / jnp.maximum(dsum, 1.0))


_ROWS_B = 1024

_finalize = pl.pallas_call(
    _finalize_body,
    grid=(NP // _ROWS_B,),
    in_specs=[
        pl.BlockSpec((NC, _ROWS_B, D), lambda i: (0, i, 0)),
        pl.BlockSpec((_ROWS_B, NC), lambda i: (i, 0)),
    ],
    out_specs=pl.BlockSpec((_ROWS_B, D), lambda i: (i, 0)),
    out_shape=jax.ShapeDtypeStruct((NP, D), jnp.float32),
)


def kernel(sender_indices, receiver_indices, type_indices,
           V_proj_sender, V_types, B_message):
    msg_p, deg_p = _sc_accumulate(sender_indices, receiver_indices,
                                  type_indices, V_proj_sender, V_types,
                                  B_message)
    deg_cols = deg_p.reshape(NC, NP // 8, 8, 16)[:, :, :, 0].reshape(NC, NP).T
    return _finalize(msg_p, deg_cols)[:N]
